# baseline (speedup 1.0000x reference)
import functools

import jax
import jax.numpy as jnp
from jax import lax
from jax.experimental import pallas as pl
from jax.experimental.pallas import tpu as pltpu
from jax.experimental.pallas import tpu_sc as plsc

E = 1_600_000
NC = 2
NS = 16
L = 16
NW = NC * NS
BLK = 128
NBLK = E // BLK
W_BLKS = NBLK // NW
W_EXTRA = NBLK - W_BLKS * NW
NB = 195
CHUNKS = W_BLKS // NB
C = NB * BLK

_f32 = jnp.float32


def _body(alpha_hbm, psi_hbm,
          a0, a1, s0, s1, ones_buf,
          in_sem0, in_sem1, out_sem0, out_sem1, ones_sem):
    wid = lax.axis_index("s") * NC + lax.axis_index("c")
    base_blk = wid * W_BLKS + jnp.minimum(wid, W_EXTRA)
    ones_f = jnp.ones((L,), _f32)

    a_bufs = (a0, a1)
    s_bufs = (s0, s1)
    in_sems = (in_sem0, in_sem1)
    out_sems = (out_sem0, out_sem1)

    def fill_ones(i):
        j = i // (BLK // L)
        k = (i % (BLK // L)) * L
        ones_buf[j, pl.ds(k, L)] = ones_f

    plsc.parallel_loop(0, NB * (BLK // L), 1, unroll=8)(fill_ones)

    def in_copy(blk, slot, nb):
        return pltpu.async_copy(
            alpha_hbm.at[pl.ds(blk * BLK, nb * BLK)],
            a_bufs[slot].at[pl.ds(0, nb * BLK)], in_sems[slot])

    def out_copies(blk, slot, nb):
        sem = out_sems[slot]
        s_src = s_bufs[slot].at[pl.ds(0, nb)]
        return (
            pltpu.async_copy(s_src, psi_hbm.at[0, pl.ds(blk, nb), 0], sem),
            pltpu.async_copy(s_src, psi_hbm.at[1, pl.ds(blk, nb), 1], sem),
        )

    def ones_copies(blk, nb):
        o_src = ones_buf.at[pl.ds(0, nb)]
        return (
            pltpu.async_copy(o_src, psi_hbm.at[0, pl.ds(blk, nb), 1],
                             ones_sem),
            pltpu.async_copy(o_src, psi_hbm.at[1, pl.ds(blk, nb), 0],
                             ones_sem),
        )

    def compute(slot, nb):
        a_buf, s_buf = a_bufs[slot], s_bufs[slot]

        def group(g):
            a = a_buf[pl.ds(g * L, L)]
            j = g // (BLK // L)
            k = (g % (BLK // L)) * L
            s_buf[j, pl.ds(k, L)] = jnp.exp(2.0 * a)

        plsc.parallel_loop(0, nb * (BLK // L), 1, unroll=4)(group)

    has_tail = wid < W_EXTRA
    tail_blk = base_blk + W_BLKS

    handles_in = [None, None]
    handles_out = [None, None]
    handles_ones = []
    handles_in[0] = in_copy(base_blk, 0, NB)
    for ci in range(CHUNKS):
        slot = ci % 2
        if ci + 1 < CHUNKS:
            handles_in[1 - slot] = in_copy(base_blk + (ci + 1) * NB,
                                           1 - slot, NB)
        handles_ones.extend(ones_copies(base_blk + ci * NB, NB))
        handles_in[slot].wait()
        if handles_out[slot] is not None:
            for h in handles_out[slot]:
                h.wait()
        compute(slot, NB)
        handles_out[slot] = out_copies(base_blk + ci * NB, slot, NB)
    for hs in handles_out:
        if hs is not None:
            for h in hs:
                h.wait()
    for h in handles_ones:
        h.wait()

    @pl.when(has_tail)
    def _tail():
        in_copy(tail_blk, 0, 1).wait()
        compute(0, 1)
        for h in ones_copies(tail_blk, 1):
            h.wait()
        for h in out_copies(tail_blk, 0, 1):
            h.wait()


@functools.cache
def _build():
    mesh = plsc.VectorSubcoreMesh(
        core_axis_name="c", subcore_axis_name="s",
        num_cores=NC, num_subcores=NS)
    return pl.kernel(
        _body,
        out_type=jax.ShapeDtypeStruct((2, NBLK, 2, BLK), _f32),
        mesh=mesh,
        compiler_params=pltpu.CompilerParams(
            needs_layout_passes=False, use_tc_tiling_on_sc=False),
        scratch_types=[
            pltpu.VMEM((C,), _f32), pltpu.VMEM((C,), _f32),
            pltpu.VMEM((NB, BLK), _f32), pltpu.VMEM((NB, BLK), _f32),
            pltpu.VMEM((NB, BLK), _f32),
            pltpu.SemaphoreType.DMA, pltpu.SemaphoreType.DMA,
            pltpu.SemaphoreType.DMA, pltpu.SemaphoreType.DMA,
            pltpu.SemaphoreType.DMA,
        ],
    )


_SIG_BS = 204_800


def _sig_body(a_ref, p_ref):
    p_ref[...] = jax.nn.sigmoid(2.0 * a_ref[...])


@functools.cache
def _build_sig():
    return pl.pallas_call(
        _sig_body,
        grid=(pl.cdiv(E, _SIG_BS),),
        in_specs=[pl.BlockSpec((_SIG_BS,), lambda i: (i,))],
        out_specs=pl.BlockSpec((_SIG_BS,), lambda i: (i,)),
        out_shape=jax.ShapeDtypeStruct((E,), _f32),
    )


def kernel(alpha_ij):
    psi4 = _build()(alpha_ij)
    p_same = _build_sig()(alpha_ij)
    psi = jnp.transpose(psi4, (1, 3, 0, 2)).reshape(E, 2, 2)
    return psi, p_same


# speedup vs baseline: 1.0318x; 1.0318x over previous
import functools

import jax
import jax.numpy as jnp
from jax import lax
from jax.experimental import pallas as pl
from jax.experimental.pallas import tpu as pltpu
from jax.experimental.pallas import tpu_sc as plsc

E = 1_600_000
NC = 2
NS = 16
L = 16
NW = NC * NS
BLK = 128
NBLK = E // BLK
W_BLKS = NBLK // NW
W_EXTRA = NBLK - W_BLKS * NW
NB = 78
CHUNKS = W_BLKS // NB
C = NB * BLK

_f32 = jnp.float32


def _body(alpha_hbm, psi_hbm,
          a0, a1, s0, s1, ones_buf,
          in_sem0, in_sem1, out_sem0, out_sem1, ones_sem):
    wid = lax.axis_index("s") * NC + lax.axis_index("c")
    base_blk = wid * W_BLKS + jnp.minimum(wid, W_EXTRA)
    ones_f = jnp.ones((L,), _f32)

    a_bufs = (a0, a1)
    s_bufs = (s0, s1)
    in_sems = (in_sem0, in_sem1)
    out_sems = (out_sem0, out_sem1)

    def fill_ones(i):
        j = i // (BLK // L)
        k = (i % (BLK // L)) * L
        ones_buf[j, pl.ds(k, L)] = ones_f

    plsc.parallel_loop(0, NB * (BLK // L), 1, unroll=8)(fill_ones)

    def in_copy(blk, slot, nb):
        return pltpu.async_copy(
            alpha_hbm.at[pl.ds(blk * BLK, nb * BLK)],
            a_bufs[slot].at[pl.ds(0, nb * BLK)], in_sems[slot])

    def out_copies(blk, slot, nb):
        sem = out_sems[slot]
        s_src = s_bufs[slot].at[pl.ds(0, nb)]
        return (
            pltpu.async_copy(s_src, psi_hbm.at[0, pl.ds(blk, nb), 0], sem),
            pltpu.async_copy(s_src, psi_hbm.at[1, pl.ds(blk, nb), 1], sem),
        )

    def ones_copies(blk, nb):
        o_src = ones_buf.at[pl.ds(0, nb)]
        return (
            pltpu.async_copy(o_src, psi_hbm.at[0, pl.ds(blk, nb), 1],
                             ones_sem),
            pltpu.async_copy(o_src, psi_hbm.at[1, pl.ds(blk, nb), 0],
                             ones_sem),
        )

    def compute(slot, nb):
        a_buf, s_buf = a_bufs[slot], s_bufs[slot]

        def group(g):
            a = a_buf[pl.ds(g * L, L)]
            j = g // (BLK // L)
            k = (g % (BLK // L)) * L
            s_buf[j, pl.ds(k, L)] = jnp.exp(2.0 * a)

        plsc.parallel_loop(0, nb * (BLK // L), 1, unroll=4)(group)

    has_tail = wid < W_EXTRA
    tail_blk = base_blk + W_BLKS

    handles_in = [None, None]
    handles_out = [None, None]
    handles_ones = []
    handles_in[0] = in_copy(base_blk, 0, NB)
    for ci in range(CHUNKS):
        slot = ci % 2
        if ci + 1 < CHUNKS:
            handles_in[1 - slot] = in_copy(base_blk + (ci + 1) * NB,
                                           1 - slot, NB)
        handles_ones.extend(ones_copies(base_blk + ci * NB, NB))
        handles_in[slot].wait()
        if handles_out[slot] is not None:
            for h in handles_out[slot]:
                h.wait()
        compute(slot, NB)
        handles_out[slot] = out_copies(base_blk + ci * NB, slot, NB)
    for hs in handles_out:
        if hs is not None:
            for h in hs:
                h.wait()
    for h in handles_ones:
        h.wait()

    @pl.when(has_tail)
    def _tail():
        in_copy(tail_blk, 0, 1).wait()
        compute(0, 1)
        for h in ones_copies(tail_blk, 1):
            h.wait()
        for h in out_copies(tail_blk, 0, 1):
            h.wait()


@functools.cache
def _build():
    mesh = plsc.VectorSubcoreMesh(
        core_axis_name="c", subcore_axis_name="s",
        num_cores=NC, num_subcores=NS)
    return pl.kernel(
        _body,
        out_type=jax.ShapeDtypeStruct((2, NBLK, 2, BLK), _f32),
        mesh=mesh,
        compiler_params=pltpu.CompilerParams(
            needs_layout_passes=False, use_tc_tiling_on_sc=False),
        scratch_types=[
            pltpu.VMEM((C,), _f32), pltpu.VMEM((C,), _f32),
            pltpu.VMEM((NB, BLK), _f32), pltpu.VMEM((NB, BLK), _f32),
            pltpu.VMEM((NB, BLK), _f32),
            pltpu.SemaphoreType.DMA, pltpu.SemaphoreType.DMA,
            pltpu.SemaphoreType.DMA, pltpu.SemaphoreType.DMA,
            pltpu.SemaphoreType.DMA,
        ],
    )


_SIG_BS = 204_800


def _sig_body(a_ref, p_ref):
    p_ref[...] = jax.nn.sigmoid(2.0 * a_ref[...])


@functools.cache
def _build_sig():
    return pl.pallas_call(
        _sig_body,
        grid=(pl.cdiv(E, _SIG_BS),),
        in_specs=[pl.BlockSpec((_SIG_BS,), lambda i: (i,))],
        out_specs=pl.BlockSpec((_SIG_BS,), lambda i: (i,)),
        out_shape=jax.ShapeDtypeStruct((E,), _f32),
    )


def kernel(alpha_ij):
    psi4 = _build()(alpha_ij)
    p_same = _build_sig()(alpha_ij)
    psi = jnp.transpose(psi4, (1, 3, 0, 2)).reshape(E, 2, 2)
    return psi, p_same


# speedup vs baseline: 1.0416x; 1.0096x over previous
import functools

import jax
import jax.numpy as jnp
from jax import lax
from jax.experimental import pallas as pl
from jax.experimental.pallas import tpu as pltpu
from jax.experimental.pallas import tpu_sc as plsc

E = 1_600_000
NC = 2
NS = 16
L = 16
NW = NC * NS
BLK = 128
NBLK = E // BLK
W_BLKS = NBLK // NW
W_EXTRA = NBLK - W_BLKS * NW
NB = 130
CHUNKS = W_BLKS // NB
C = NB * BLK

_f32 = jnp.float32


def _body(alpha_hbm, psi_hbm,
          a0, a1, s0, s1, ones_buf,
          in_sem0, in_sem1, out_sem0, out_sem1, ones_sem):
    wid = lax.axis_index("s") * NC + lax.axis_index("c")
    base_blk = wid * W_BLKS + jnp.minimum(wid, W_EXTRA)
    ones_f = jnp.ones((L,), _f32)

    a_bufs = (a0, a1)
    s_bufs = (s0, s1)
    in_sems = (in_sem0, in_sem1)
    out_sems = (out_sem0, out_sem1)

    def fill_ones(i):
        j = i // (BLK // L)
        k = (i % (BLK // L)) * L
        ones_buf[j, pl.ds(k, L)] = ones_f

    plsc.parallel_loop(0, NB * (BLK // L), 1, unroll=8)(fill_ones)

    def in_copy(blk, slot, nb):
        return pltpu.async_copy(
            alpha_hbm.at[pl.ds(blk * BLK, nb * BLK)],
            a_bufs[slot].at[pl.ds(0, nb * BLK)], in_sems[slot])

    def out_copies(blk, slot, nb):
        sem = out_sems[slot]
        s_src = s_bufs[slot].at[pl.ds(0, nb)]
        return (
            pltpu.async_copy(s_src, psi_hbm.at[0, pl.ds(blk, nb), 0], sem),
            pltpu.async_copy(s_src, psi_hbm.at[1, pl.ds(blk, nb), 1], sem),
        )

    def ones_copies(blk, nb):
        o_src = ones_buf.at[pl.ds(0, nb)]
        return (
            pltpu.async_copy(o_src, psi_hbm.at[0, pl.ds(blk, nb), 1],
                             ones_sem),
            pltpu.async_copy(o_src, psi_hbm.at[1, pl.ds(blk, nb), 0],
                             ones_sem),
        )

    def compute(slot, nb):
        a_buf, s_buf = a_bufs[slot], s_bufs[slot]

        def group(g):
            a = a_buf[pl.ds(g * L, L)]
            j = g // (BLK // L)
            k = (g % (BLK // L)) * L
            s_buf[j, pl.ds(k, L)] = jnp.exp(2.0 * a)

        plsc.parallel_loop(0, nb * (BLK // L), 1, unroll=8)(group)

    has_tail = wid < W_EXTRA
    tail_blk = base_blk + W_BLKS

    handles_in = [None, None]
    handles_out = [None, None]
    handles_ones = []
    handles_in[0] = in_copy(base_blk, 0, NB)
    for ci in range(CHUNKS):
        slot = ci % 2
        if ci + 1 < CHUNKS:
            handles_in[1 - slot] = in_copy(base_blk + (ci + 1) * NB,
                                           1 - slot, NB)
        handles_ones.extend(ones_copies(base_blk + ci * NB, NB))
        handles_in[slot].wait()
        if handles_out[slot] is not None:
            for h in handles_out[slot]:
                h.wait()
        compute(slot, NB)
        handles_out[slot] = out_copies(base_blk + ci * NB, slot, NB)
    for hs in handles_out:
        if hs is not None:
            for h in hs:
                h.wait()
    for h in handles_ones:
        h.wait()

    @pl.when(has_tail)
    def _tail():
        in_copy(tail_blk, 0, 1).wait()
        compute(0, 1)
        for h in ones_copies(tail_blk, 1):
            h.wait()
        for h in out_copies(tail_blk, 0, 1):
            h.wait()


@functools.cache
def _build():
    mesh = plsc.VectorSubcoreMesh(
        core_axis_name="c", subcore_axis_name="s",
        num_cores=NC, num_subcores=NS)
    return pl.kernel(
        _body,
        out_type=jax.ShapeDtypeStruct((2, NBLK, 2, BLK), _f32),
        mesh=mesh,
        compiler_params=pltpu.CompilerParams(
            needs_layout_passes=False, use_tc_tiling_on_sc=False),
        scratch_types=[
            pltpu.VMEM((C,), _f32), pltpu.VMEM((C,), _f32),
            pltpu.VMEM((NB, BLK), _f32), pltpu.VMEM((NB, BLK), _f32),
            pltpu.VMEM((NB, BLK), _f32),
            pltpu.SemaphoreType.DMA, pltpu.SemaphoreType.DMA,
            pltpu.SemaphoreType.DMA, pltpu.SemaphoreType.DMA,
            pltpu.SemaphoreType.DMA,
        ],
    )


_SIG_BS = 204_800


def _sig_body(a_ref, p_ref):
    p_ref[...] = jax.nn.sigmoid(2.0 * a_ref[...])


@functools.cache
def _build_sig():
    return pl.pallas_call(
        _sig_body,
        grid=(pl.cdiv(E, _SIG_BS),),
        in_specs=[pl.BlockSpec((_SIG_BS,), lambda i: (i,))],
        out_specs=pl.BlockSpec((_SIG_BS,), lambda i: (i,)),
        out_shape=jax.ShapeDtypeStruct((E,), _f32),
    )


def kernel(alpha_ij):
    psi4 = _build()(alpha_ij)
    p_same = _build_sig()(alpha_ij)
    psi = jnp.transpose(psi4, (1, 3, 0, 2)).reshape(E, 2, 2)
    return psi, p_same
